# baseline (speedup 1.0000x reference)
import functools
import jax
import jax.numpy as jnp
from jax import lax
from jax.experimental import pallas as pl
from jax.experimental.pallas import tpu as pltpu
from jax.experimental.pallas import tpu_sc as plsc

N = 10000
E = 160000
D_FEAT = 128
EMB = 16
R = 8
L = 5

NC = 2
NS = 16
NW = NC * NS
CH = 500
NCH = E // CH
NCHT = NCH // NW
ZB = 640

_SC_MESH = plsc.VectorSubcoreMesh(core_axis_name="c", subcore_axis_name="s")


def _zero_acc(acc, zbuf, s, rpt, zb):
    @pl.loop(0, ZB)
    def _(i):
        zbuf[i, :] = jnp.zeros((EMB,), jnp.float32)

    @pl.loop(0, rpt, step=zb)
    def _(i):
        pltpu.sync_copy(zbuf.at[pl.ds(0, zb)], acc.at[pl.ds(s * rpt + i, zb)])


def _sc_scatter_body(SP, rows_hbm, idx_hbm, out_hbm, idxv, rb0, rb1, zbuf,
                     acc, sem0, sem1):
    c = lax.axis_index("c")
    s = lax.axis_index("s")
    w = c * NS + s
    rpt = SP // NS
    _zero_acc(acc, zbuf, s, rpt, ZB)
    plsc.subcore_barrier()
    pltpu.sync_copy(idx_hbm.at[pl.ds(w * NCHT, NCHT)], idxv)
    pltpu.async_copy(rows_hbm.at[pl.ds(w * NCHT * CH, CH)], rb0, sem0)

    @pl.loop(0, NCHT, step=2)
    def _(j):
        ch = (w * NCHT + j) * CH
        pltpu.async_copy(rows_hbm.at[pl.ds(ch + CH, CH)], rb1, sem1)
        pltpu.make_async_copy(rows_hbm.at[pl.ds(ch, CH)], rb0, sem0).wait()
        pltpu.sync_copy(rb0, acc.at[idxv.at[j]], add=True)

        @pl.when(j + 2 < NCHT)
        def _():
            pltpu.async_copy(rows_hbm.at[pl.ds(ch + 2 * CH, CH)], rb0, sem0)

        pltpu.make_async_copy(rows_hbm.at[pl.ds(ch + CH, CH)], rb1, sem1).wait()
        pltpu.sync_copy(rb1, acc.at[idxv.at[j + 1]], add=True)

    plsc.subcore_barrier()

    @pl.loop(0, rpt, step=ZB)
    def _(i):
        pltpu.sync_copy(acc.at[pl.ds(s * rpt + i, ZB)],
                        out_hbm.at[c].at[pl.ds(s * rpt + i, ZB)])


def _sc_scatter_add(rows, idx2, SP):
    f = pl.kernel(
        functools.partial(_sc_scatter_body, SP),
        out_type=jax.ShapeDtypeStruct((NC, SP, EMB), jnp.float32),
        mesh=_SC_MESH,
        compiler_params=pltpu.CompilerParams(use_tc_tiling_on_sc=False),
        scratch_types=[
            pltpu.VMEM((NCHT, CH), jnp.int32),
            pltpu.VMEM((CH, EMB), jnp.float32),
            pltpu.VMEM((CH, EMB), jnp.float32),
            pltpu.VMEM((ZB, EMB), jnp.float32),
            pltpu.VMEM_SHARED((SP, EMB), jnp.float32),
            pltpu.SemaphoreType.DMA,
            pltpu.SemaphoreType.DMA,
        ],
    )
    return f(rows, idx2)


HNR = N * R // 2
SP_HALF = HNR + EMB
ZB_SR = 500


def _sc_sr_body(rows_hbm, idx_hbm, out_hbm, idxv, rb0, rb1, zbuf, acc,
                sem0, sem1):
    c = lax.axis_index("c")
    s = lax.axis_index("s")
    npc = NCH // NS
    rpt = HNR // NS
    _zero_acc(acc, zbuf, s, rpt, ZB_SR)

    @pl.when(s == 0)
    def _():
        pltpu.sync_copy(zbuf.at[pl.ds(0, EMB)], acc.at[pl.ds(HNR, EMB)])

    plsc.subcore_barrier()
    pltpu.sync_copy(idx_hbm.at[c].at[pl.ds(s * npc, npc)], idxv)
    pltpu.async_copy(rows_hbm.at[pl.ds(s * npc * CH, CH)], rb0, sem0)

    @pl.loop(0, npc, step=2)
    def _(j):
        ch = (s * npc + j) * CH
        pltpu.async_copy(rows_hbm.at[pl.ds(ch + CH, CH)], rb1, sem1)
        pltpu.make_async_copy(rows_hbm.at[pl.ds(ch, CH)], rb0, sem0).wait()
        pltpu.sync_copy(rb0, acc.at[idxv.at[j]], add=True)

        @pl.when(j + 2 < npc)
        def _():
            pltpu.async_copy(rows_hbm.at[pl.ds(ch + 2 * CH, CH)], rb0, sem0)

        pltpu.make_async_copy(rows_hbm.at[pl.ds(ch + CH, CH)], rb1, sem1).wait()
        pltpu.sync_copy(rb1, acc.at[idxv.at[j + 1]], add=True)

    plsc.subcore_barrier()

    @pl.loop(0, rpt, step=ZB_SR)
    def _(i):
        pltpu.sync_copy(acc.at[pl.ds(s * rpt + i, ZB_SR)],
                        out_hbm.at[c].at[pl.ds(s * rpt + i, ZB_SR)])


def _sc_sr_scatter(rows, idx_ranged):
    npc = NCH // NS
    f = pl.kernel(
        _sc_sr_body,
        out_type=jax.ShapeDtypeStruct((NC, HNR, EMB), jnp.float32),
        mesh=_SC_MESH,
        compiler_params=pltpu.CompilerParams(use_tc_tiling_on_sc=False),
        scratch_types=[
            pltpu.VMEM((npc, CH), jnp.int32),
            pltpu.VMEM((CH, EMB), jnp.float32),
            pltpu.VMEM((CH, EMB), jnp.float32),
            pltpu.VMEM((ZB, EMB), jnp.float32),
            pltpu.VMEM_SHARED((SP_HALF, EMB), jnp.float32),
            pltpu.SemaphoreType.DMA,
            pltpu.SemaphoreType.DMA,
        ],
    )
    return f(rows, idx_ranged)


def _sc_gather_body(h_hbm, idx_hbm, out_hbm, idxv, rb0, rb1, sem0, sem1):
    c = lax.axis_index("c")
    s = lax.axis_index("s")
    w = c * NS + s
    pltpu.sync_copy(idx_hbm.at[pl.ds(w * NCHT, NCHT)], idxv)
    pltpu.async_copy(h_hbm.at[idxv.at[0]], rb0, sem0)

    @pl.loop(0, NCHT, step=2)
    def _(j):
        ch = (w * NCHT + j) * CH
        pltpu.async_copy(h_hbm.at[idxv.at[j + 1]], rb1, sem1)
        pltpu.make_async_copy(h_hbm.at[idxv.at[j]], rb0, sem0).wait()
        pltpu.sync_copy(rb0, out_hbm.at[pl.ds(ch, CH)])

        @pl.when(j + 2 < NCHT)
        def _():
            pltpu.async_copy(h_hbm.at[idxv.at[j + 2]], rb0, sem0)

        pltpu.make_async_copy(h_hbm.at[idxv.at[j + 1]], rb1, sem1).wait()
        pltpu.sync_copy(rb1, out_hbm.at[pl.ds(ch + CH, CH)])


def _sc_gather(h, src2):
    f = pl.kernel(
        _sc_gather_body,
        out_type=jax.ShapeDtypeStruct((E, EMB), jnp.float32),
        mesh=_SC_MESH,
        compiler_params=pltpu.CompilerParams(use_tc_tiling_on_sc=False),
        scratch_types=[
            pltpu.VMEM((NCHT, CH), jnp.int32),
            pltpu.VMEM((CH, EMB), jnp.float32),
            pltpu.VMEM((CH, EMB), jnp.float32),
            pltpu.SemaphoreType.DMA,
            pltpu.SemaphoreType.DMA,
        ],
    )
    return f(h, src2)


def _proj_body(x_ref, w_ref, b_ref, o_ref):
    o_ref[...] = jax.nn.relu(
        jnp.dot(x_ref[...], w_ref[...], preferred_element_type=jnp.float32)
        + b_ref[...]
    )


def _node_proj(x, W_in, b_in):
    Tn = 2048
    grid = (pl.cdiv(N, Tn),)
    return pl.pallas_call(
        _proj_body,
        grid=grid,
        in_specs=[
            pl.BlockSpec((Tn, D_FEAT), lambda i: (i, 0)),
            pl.BlockSpec((D_FEAT, EMB), lambda i: (0, 0)),
            pl.BlockSpec((1, EMB), lambda i: (0, 0)),
        ],
        out_specs=pl.BlockSpec((Tn, EMB), lambda i: (i, 0)),
        out_shape=jax.ShapeDtypeStruct((N, EMB), jnp.float32),
    )(x, W_in, b_in.reshape(1, EMB))


def _edge_prep_body(attr_ref, w1r_ref, w1d_ref, b1_ref, hid_ref, oh_ref):
    attr = attr_ref[...]
    dist = attr[:, 0:1]
    etype = attr[:, 1].astype(jnp.int32)
    lane = jax.lax.broadcasted_iota(jnp.int32, attr.shape[:1] + (EMB,), 1)
    oh = (lane == etype[:, None]).astype(jnp.float32)
    hid = jax.nn.relu(
        dist * w1d_ref[...]
        + jnp.dot(oh, w1r_ref[...], preferred_element_type=jnp.float32)
        + b1_ref[...]
    )
    hid_ref[...] = hid
    oh_ref[...] = oh


def _edge_prep(edge_attr, mlp_w1, mlp_b1):
    w1_rel = jnp.zeros((EMB, EMB), jnp.float32).at[:R].set(mlp_w1[1 : 1 + R])
    Te = 4000
    grid = (pl.cdiv(E, Te),)
    return pl.pallas_call(
        _edge_prep_body,
        grid=grid,
        in_specs=[
            pl.BlockSpec((Te, 2), lambda i: (i, 0)),
            pl.BlockSpec((EMB, EMB), lambda i: (0, 0)),
            pl.BlockSpec((1, EMB), lambda i: (0, 0)),
            pl.BlockSpec((1, EMB), lambda i: (0, 0)),
        ],
        out_specs=[
            pl.BlockSpec((Te, EMB), lambda i: (i, 0)),
            pl.BlockSpec((Te, EMB), lambda i: (i, 0)),
        ],
        out_shape=[
            jax.ShapeDtypeStruct((E, EMB), jnp.float32),
            jax.ShapeDtypeStruct((E, EMB), jnp.float32),
        ],
    )(edge_attr, w1_rel, mlp_w1[0].reshape(1, EMB), mlp_b1.reshape(1, EMB))


def _msg_body(hs_ref, hid_ref, k1_ref, k2_ref, w2p_ref, b2_ref, o_ref):
    hs = hs_ref[...]
    a = jnp.dot(hs, k1_ref[...], preferred_element_type=jnp.float32)
    b = jnp.dot(hid_ref[...], k2_ref[...], preferred_element_type=jnp.float32)
    o_ref[...] = (
        jnp.dot(a * b, w2p_ref[...], preferred_element_type=jnp.float32)
        + jnp.dot(hs, b2_ref[...], preferred_element_type=jnp.float32)
    )


E8 = E // 8


def _msg(h_src_p, hidden_p, K1B, K2B, W2PB, B2B):
    Te = 1000
    grid = (pl.cdiv(E8, Te),)
    return pl.pallas_call(
        _msg_body,
        grid=grid,
        in_specs=[
            pl.BlockSpec((Te, 128), lambda i: (i, 0)),
            pl.BlockSpec((Te, 128), lambda i: (i, 0)),
            pl.BlockSpec((128, 2048), lambda i: (0, 0)),
            pl.BlockSpec((128, 2048), lambda i: (0, 0)),
            pl.BlockSpec((2048, 128), lambda i: (0, 0)),
            pl.BlockSpec((128, 128), lambda i: (0, 0)),
        ],
        out_specs=pl.BlockSpec((Te, 128), lambda i: (i, 0)),
        out_shape=jax.ShapeDtypeStruct((E8, 128), jnp.float32),
    )(h_src_p, hidden_p, K1B, K2B, W2PB, B2B)


_TN = 1000


def _update_body(
    h_ref, sr_ref, ms0_ref, ms1_ref, cnt0_ref, cnt1_ref,
    wst_ref, root_ref, bias_ref, nnroot_ref, nnb_ref, o_ref
):
    h = h_ref[...]
    cnt = cnt0_ref[...] + cnt1_ref[...]
    rmask = jax.lax.broadcasted_iota(jnp.int32, cnt.shape, 1) < R
    recip = 1.0 / jnp.maximum(cnt, 1.0)
    sr3 = sr_ref[0].reshape(_TN, R, EMB)
    acc = jnp.dot(h, root_ref[...], preferred_element_type=jnp.float32)
    for r in range(R):
        acc = acc + jnp.dot(sr3[:, r, :] * recip[:, r : r + 1], wst_ref[r],
                            preferred_element_type=jnp.float32)
    h_disc = jax.nn.relu(acc + bias_ref[...])
    cnt_all = jnp.sum(jnp.where(rmask, cnt, 0.0), axis=1, keepdims=True)
    rall = 1.0 / jnp.maximum(cnt_all, 1.0)
    h_cont = jax.nn.relu(
        (ms0_ref[...] + ms1_ref[...]) * rall
        + jnp.dot(h, nnroot_ref[...], preferred_element_type=jnp.float32)
        + nnb_ref[...]
    )
    o_ref[...] = h + h_disc + h_cont


def _node_update(h, SRP, MS0, MS1, CNT0, CNT1, Wstack, root, bias,
                 nnroot, nnb):
    Tn = _TN
    nb = N // (2 * Tn)
    grid = (pl.cdiv(N, Tn),)
    return pl.pallas_call(
        _update_body,
        grid=grid,
        in_specs=[
            pl.BlockSpec((Tn, EMB), lambda i: (i, 0)),
            pl.BlockSpec((1, Tn * R, EMB), lambda i: (i // nb, i % nb, 0)),
            pl.BlockSpec((Tn, EMB), lambda i: (i, 0)),
            pl.BlockSpec((Tn, EMB), lambda i: (i, 0)),
            pl.BlockSpec((Tn, EMB), lambda i: (i, 0)),
            pl.BlockSpec((Tn, EMB), lambda i: (i, 0)),
            pl.BlockSpec((R, EMB, EMB), lambda i: (0, 0, 0)),
            pl.BlockSpec((EMB, EMB), lambda i: (0, 0)),
            pl.BlockSpec((1, EMB), lambda i: (0, 0)),
            pl.BlockSpec((EMB, EMB), lambda i: (0, 0)),
            pl.BlockSpec((1, EMB), lambda i: (0, 0)),
        ],
        out_specs=pl.BlockSpec((Tn, EMB), lambda i: (i, 0)),
        out_shape=jax.ShapeDtypeStruct((N, EMB), jnp.float32),
    )(h, SRP, MS0, MS1, CNT0, CNT1, Wstack, root,
      bias.reshape(1, EMB), nnroot, nnb.reshape(1, EMB))


def kernel(x, edge_index, edge_attr, W_in, b_in, rgcn_weight, rgcn_root,
           rgcn_bias, nn_root, nn_bias, mlp_w1, mlp_b1, mlp_w2, mlp_b2):
    src = edge_index[0]
    dst = edge_index[1]
    etype = edge_attr[:, 1].astype(jnp.int32)
    seg_rel = dst * R + etype

    col = jnp.arange(16 * EMB)
    K1 = (jnp.arange(EMB)[:, None] == (col // EMB)[None, :]).astype(jnp.float32)
    K2 = (jnp.arange(EMB)[:, None] == (col % EMB)[None, :]).astype(jnp.float32)
    W2p = mlp_w2.reshape(EMB, EMB, EMB).transpose(1, 0, 2).reshape(EMB * EMB, EMB)
    B2 = mlp_b2.reshape(EMB, EMB)
    I8 = jnp.eye(8, dtype=jnp.float32)
    K1B = jnp.kron(I8, K1)
    K2B = jnp.kron(I8, K2)
    W2PB = jnp.kron(I8, W2p)
    B2B = jnp.kron(I8, B2)

    h = _node_proj(x, W_in, b_in)
    hidden, onehot = _edge_prep(edge_attr, mlp_w1, mlp_b1)

    SP_N = 10240
    src2 = src.reshape(NCH, CH)
    idx_ranged = jnp.stack([
        jnp.where(seg_rel < HNR, seg_rel, HNR),
        jnp.where(seg_rel >= HNR, seg_rel - HNR, HNR),
    ]).reshape(NC, NCH, CH)

    dst2 = dst.reshape(NCH, CH)
    cntp = _sc_scatter_add(onehot, dst2, SP_N)
    CNT0, CNT1 = cntp[0, :N], cntp[1, :N]
    hidden_p = hidden.reshape(E8, 128)

    for l in range(L):
        h_src = _sc_gather(h, src2)
        srp = _sc_sr_scatter(h_src, idx_ranged)
        msg_p = _msg(h_src.reshape(E8, 128), hidden_p, K1B, K2B, W2PB, B2B)
        dep = (srp[0, 0] * 0.0).astype(jnp.int32)[0]
        msp = _sc_scatter_add(msg_p.reshape(E, EMB), dst2 + dep, SP_N)
        MS0, MS1 = msp[0, :N], msp[1, :N]
        h = _node_update(h, srp, MS0, MS1, CNT0, CNT1, rgcn_weight[l],
                         rgcn_root[l], rgcn_bias[l], nn_root[l], nn_bias[l])
    return h


# speedup vs baseline: 1.1415x; 1.1415x over previous
import functools
import jax
import jax.numpy as jnp
from jax import lax
from jax.experimental import pallas as pl
from jax.experimental.pallas import tpu as pltpu
from jax.experimental.pallas import tpu_sc as plsc

N = 10000
E = 160000
D_FEAT = 128
EMB = 16
R = 8
L = 5

NC = 2
NS = 16
NW = NC * NS
CH = 500
NCH = E // CH
NCHT = NCH // NW
ZB = 640

_SC_MESH = plsc.VectorSubcoreMesh(core_axis_name="c", subcore_axis_name="s")


def _zero_acc(acc, zbuf, s, rpt, zb):
    @pl.loop(0, ZB)
    def _(i):
        zbuf[i, :] = jnp.zeros((EMB,), jnp.float32)

    @pl.loop(0, rpt, step=zb)
    def _(i):
        pltpu.sync_copy(zbuf.at[pl.ds(0, zb)], acc.at[pl.ds(s * rpt + i, zb)])


def _sc_scatter_body(SP, rows_hbm, idx_hbm, out_hbm, idxv, rb0, rb1, zbuf,
                     acc, sem0, sem1):
    c = lax.axis_index("c")
    s = lax.axis_index("s")
    w = c * NS + s
    rpt = SP // NS
    _zero_acc(acc, zbuf, s, rpt, ZB)
    plsc.subcore_barrier()
    pltpu.sync_copy(idx_hbm.at[pl.ds(w * NCHT, NCHT)], idxv)
    pltpu.async_copy(rows_hbm.at[pl.ds(w * NCHT * CH, CH)], rb0, sem0)

    @pl.loop(0, NCHT, step=2)
    def _(j):
        ch = (w * NCHT + j) * CH
        pltpu.async_copy(rows_hbm.at[pl.ds(ch + CH, CH)], rb1, sem1)
        pltpu.make_async_copy(rows_hbm.at[pl.ds(ch, CH)], rb0, sem0).wait()
        pltpu.sync_copy(rb0, acc.at[idxv.at[j]], add=True)

        @pl.when(j + 2 < NCHT)
        def _():
            pltpu.async_copy(rows_hbm.at[pl.ds(ch + 2 * CH, CH)], rb0, sem0)

        pltpu.make_async_copy(rows_hbm.at[pl.ds(ch + CH, CH)], rb1, sem1).wait()
        pltpu.sync_copy(rb1, acc.at[idxv.at[j + 1]], add=True)

    plsc.subcore_barrier()

    @pl.loop(0, rpt, step=ZB)
    def _(i):
        pltpu.sync_copy(acc.at[pl.ds(s * rpt + i, ZB)],
                        out_hbm.at[c].at[pl.ds(s * rpt + i, ZB)])


def _sc_scatter_add(rows, idx2, SP):
    f = pl.kernel(
        functools.partial(_sc_scatter_body, SP),
        out_type=jax.ShapeDtypeStruct((NC, SP, EMB), jnp.float32),
        mesh=_SC_MESH,
        compiler_params=pltpu.CompilerParams(use_tc_tiling_on_sc=False),
        scratch_types=[
            pltpu.VMEM((NCHT, CH), jnp.int32),
            pltpu.VMEM((CH, EMB), jnp.float32),
            pltpu.VMEM((CH, EMB), jnp.float32),
            pltpu.VMEM((ZB, EMB), jnp.float32),
            pltpu.VMEM_SHARED((SP, EMB), jnp.float32),
            pltpu.SemaphoreType.DMA,
            pltpu.SemaphoreType.DMA,
        ],
    )
    return f(rows, idx2)


HNR = N * R // 2
SP_HALF = HNR + EMB
ZB_SR = 500


def _sc_sr_body(rows_hbm, idx_hbm, out_hbm, idxv, rb0, rb1, zbuf, acc,
                sem0, sem1):
    c = lax.axis_index("c")
    s = lax.axis_index("s")
    npc = NCH // NS
    rpt = HNR // NS
    _zero_acc(acc, zbuf, s, rpt, ZB_SR)

    @pl.when(s == 0)
    def _():
        pltpu.sync_copy(zbuf.at[pl.ds(0, EMB)], acc.at[pl.ds(HNR, EMB)])

    plsc.subcore_barrier()
    pltpu.sync_copy(idx_hbm.at[c].at[pl.ds(s * npc, npc)], idxv)
    pltpu.async_copy(rows_hbm.at[pl.ds(s * npc * CH, CH)], rb0, sem0)

    @pl.loop(0, npc, step=2)
    def _(j):
        ch = (s * npc + j) * CH
        pltpu.async_copy(rows_hbm.at[pl.ds(ch + CH, CH)], rb1, sem1)
        pltpu.make_async_copy(rows_hbm.at[pl.ds(ch, CH)], rb0, sem0).wait()
        pltpu.sync_copy(rb0, acc.at[idxv.at[j]], add=True)

        @pl.when(j + 2 < npc)
        def _():
            pltpu.async_copy(rows_hbm.at[pl.ds(ch + 2 * CH, CH)], rb0, sem0)

        pltpu.make_async_copy(rows_hbm.at[pl.ds(ch + CH, CH)], rb1, sem1).wait()
        pltpu.sync_copy(rb1, acc.at[idxv.at[j + 1]], add=True)

    plsc.subcore_barrier()

    @pl.loop(0, rpt, step=ZB_SR)
    def _(i):
        pltpu.sync_copy(acc.at[pl.ds(s * rpt + i, ZB_SR)],
                        out_hbm.at[c].at[pl.ds(s * rpt + i, ZB_SR)])


def _sc_sr_scatter(rows, idx_ranged):
    npc = NCH // NS
    f = pl.kernel(
        _sc_sr_body,
        out_type=jax.ShapeDtypeStruct((NC, HNR, EMB), jnp.float32),
        mesh=_SC_MESH,
        compiler_params=pltpu.CompilerParams(use_tc_tiling_on_sc=False),
        scratch_types=[
            pltpu.VMEM((npc, CH), jnp.int32),
            pltpu.VMEM((CH, EMB), jnp.float32),
            pltpu.VMEM((CH, EMB), jnp.float32),
            pltpu.VMEM((ZB, EMB), jnp.float32),
            pltpu.VMEM_SHARED((SP_HALF, EMB), jnp.float32),
            pltpu.SemaphoreType.DMA,
            pltpu.SemaphoreType.DMA,
        ],
    )
    return f(rows, idx_ranged)


def _sc_gather_body(h_hbm, idx_hbm, out_hbm, idxv, rb0, rb1, sem0, sem1):
    c = lax.axis_index("c")
    s = lax.axis_index("s")
    w = c * NS + s
    pltpu.sync_copy(idx_hbm.at[pl.ds(w * NCHT, NCHT)], idxv)
    pltpu.async_copy(h_hbm.at[idxv.at[0]], rb0, sem0)

    @pl.loop(0, NCHT, step=2)
    def _(j):
        ch = (w * NCHT + j) * CH
        pltpu.async_copy(h_hbm.at[idxv.at[j + 1]], rb1, sem1)
        pltpu.make_async_copy(h_hbm.at[idxv.at[j]], rb0, sem0).wait()
        pltpu.sync_copy(rb0, out_hbm.at[pl.ds(ch, CH)])

        @pl.when(j + 2 < NCHT)
        def _():
            pltpu.async_copy(h_hbm.at[idxv.at[j + 2]], rb0, sem0)

        pltpu.make_async_copy(h_hbm.at[idxv.at[j + 1]], rb1, sem1).wait()
        pltpu.sync_copy(rb1, out_hbm.at[pl.ds(ch + CH, CH)])


def _sc_gather(h, src2):
    f = pl.kernel(
        _sc_gather_body,
        out_type=jax.ShapeDtypeStruct((E, EMB), jnp.float32),
        mesh=_SC_MESH,
        compiler_params=pltpu.CompilerParams(use_tc_tiling_on_sc=False),
        scratch_types=[
            pltpu.VMEM((NCHT, CH), jnp.int32),
            pltpu.VMEM((CH, EMB), jnp.float32),
            pltpu.VMEM((CH, EMB), jnp.float32),
            pltpu.SemaphoreType.DMA,
            pltpu.SemaphoreType.DMA,
        ],
    )
    return f(h, src2)


def _proj_body(x_ref, w_ref, b_ref, o_ref):
    o_ref[...] = jax.nn.relu(
        jnp.dot(x_ref[...], w_ref[...], preferred_element_type=jnp.float32)
        + b_ref[...]
    )


def _node_proj(x, W_in, b_in):
    Tn = 2048
    grid = (pl.cdiv(N, Tn),)
    return pl.pallas_call(
        _proj_body,
        grid=grid,
        in_specs=[
            pl.BlockSpec((Tn, D_FEAT), lambda i: (i, 0)),
            pl.BlockSpec((D_FEAT, EMB), lambda i: (0, 0)),
            pl.BlockSpec((1, EMB), lambda i: (0, 0)),
        ],
        out_specs=pl.BlockSpec((Tn, EMB), lambda i: (i, 0)),
        out_shape=jax.ShapeDtypeStruct((N, EMB), jnp.float32),
    )(x, W_in, b_in.reshape(1, EMB))


def _edge_prep_body(attr_ref, w1r_ref, w1d_ref, b1_ref, hid_ref, oh_ref):
    attr = attr_ref[...]
    dist = attr[:, 0:1]
    etype = attr[:, 1].astype(jnp.int32)
    lane = jax.lax.broadcasted_iota(jnp.int32, attr.shape[:1] + (EMB,), 1)
    oh = (lane == etype[:, None]).astype(jnp.float32)
    hid = jax.nn.relu(
        dist * w1d_ref[...]
        + jnp.dot(oh, w1r_ref[...], preferred_element_type=jnp.float32)
        + b1_ref[...]
    )
    hid_ref[...] = hid
    oh_ref[...] = oh


def _edge_prep(edge_attr, mlp_w1, mlp_b1):
    w1_rel = jnp.zeros((EMB, EMB), jnp.float32).at[:R].set(mlp_w1[1 : 1 + R])
    Te = 4000
    grid = (pl.cdiv(E, Te),)
    return pl.pallas_call(
        _edge_prep_body,
        grid=grid,
        in_specs=[
            pl.BlockSpec((Te, 2), lambda i: (i, 0)),
            pl.BlockSpec((EMB, EMB), lambda i: (0, 0)),
            pl.BlockSpec((1, EMB), lambda i: (0, 0)),
            pl.BlockSpec((1, EMB), lambda i: (0, 0)),
        ],
        out_specs=[
            pl.BlockSpec((Te, EMB), lambda i: (i, 0)),
            pl.BlockSpec((Te, EMB), lambda i: (i, 0)),
        ],
        out_shape=[
            jax.ShapeDtypeStruct((E, EMB), jnp.float32),
            jax.ShapeDtypeStruct((E, EMB), jnp.float32),
        ],
    )(edge_attr, w1_rel, mlp_w1[0].reshape(1, EMB), mlp_b1.reshape(1, EMB))


def _msg_body(hs_ref, hid_ref, k1_ref, k2_ref, w2p_ref, b2_ref, o_ref):
    hs = hs_ref[...]
    a = jnp.dot(hs, k1_ref[...], preferred_element_type=jnp.float32)
    b = jnp.dot(hid_ref[...], k2_ref[...], preferred_element_type=jnp.float32)
    o_ref[...] = (
        jnp.dot(a * b, w2p_ref[...], preferred_element_type=jnp.float32)
        + jnp.dot(hs, b2_ref[...], preferred_element_type=jnp.float32)
    )


E8 = E // 8


def _msg(h_src_p, hidden_p, K1B, K2B, W2PB, B2B):
    Te = 1000
    grid = (pl.cdiv(E8, Te),)
    return pl.pallas_call(
        _msg_body,
        grid=grid,
        in_specs=[
            pl.BlockSpec((Te, 128), lambda i: (i, 0)),
            pl.BlockSpec((Te, 128), lambda i: (i, 0)),
            pl.BlockSpec((128, 2048), lambda i: (0, 0)),
            pl.BlockSpec((128, 2048), lambda i: (0, 0)),
            pl.BlockSpec((2048, 128), lambda i: (0, 0)),
            pl.BlockSpec((128, 128), lambda i: (0, 0)),
        ],
        out_specs=pl.BlockSpec((Te, 128), lambda i: (i, 0)),
        out_shape=jax.ShapeDtypeStruct((E8, 128), jnp.float32),
    )(h_src_p, hidden_p, K1B, K2B, W2PB, B2B)


def _update_body(
    h_ref, sr_ref, ms0_ref, ms1_ref, cnt0_ref, cnt1_ref, k3_ref,
    wst_ref, root_ref, bias_ref, nnroot_ref, nnb_ref, o_ref
):
    h = h_ref[...]
    cnt = cnt0_ref[...] + cnt1_ref[...]
    rmask = jax.lax.broadcasted_iota(jnp.int32, cnt.shape, 1) < R
    recip = 1.0 / jnp.maximum(cnt, 1.0)
    rcnt_rel = jnp.dot(recip, k3_ref[...], preferred_element_type=jnp.float32)
    mean_rel = sr_ref[...] * rcnt_rel
    h_disc = jax.nn.relu(
        jnp.dot(mean_rel, wst_ref[...], preferred_element_type=jnp.float32)
        + jnp.dot(h, root_ref[...], preferred_element_type=jnp.float32)
        + bias_ref[...]
    )
    cnt_all = jnp.sum(jnp.where(rmask, cnt, 0.0), axis=1, keepdims=True)
    rall = 1.0 / jnp.maximum(cnt_all, 1.0)
    h_cont = jax.nn.relu(
        (ms0_ref[...] + ms1_ref[...]) * rall
        + jnp.dot(h, nnroot_ref[...], preferred_element_type=jnp.float32)
        + nnb_ref[...]
    )
    o_ref[...] = h + h_disc + h_cont


def _node_update(h, SR, MS0, MS1, CNT0, CNT1, K3, Wstack, root, bias,
                 nnroot, nnb):
    Tn = 2048
    grid = (pl.cdiv(N, Tn),)
    return pl.pallas_call(
        _update_body,
        grid=grid,
        in_specs=[
            pl.BlockSpec((Tn, EMB), lambda i: (i, 0)),
            pl.BlockSpec((Tn, R * EMB), lambda i: (i, 0)),
            pl.BlockSpec((Tn, EMB), lambda i: (i, 0)),
            pl.BlockSpec((Tn, EMB), lambda i: (i, 0)),
            pl.BlockSpec((Tn, EMB), lambda i: (i, 0)),
            pl.BlockSpec((Tn, EMB), lambda i: (i, 0)),
            pl.BlockSpec((EMB, R * EMB), lambda i: (0, 0)),
            pl.BlockSpec((R * EMB, EMB), lambda i: (0, 0)),
            pl.BlockSpec((EMB, EMB), lambda i: (0, 0)),
            pl.BlockSpec((1, EMB), lambda i: (0, 0)),
            pl.BlockSpec((EMB, EMB), lambda i: (0, 0)),
            pl.BlockSpec((1, EMB), lambda i: (0, 0)),
        ],
        out_specs=pl.BlockSpec((Tn, EMB), lambda i: (i, 0)),
        out_shape=jax.ShapeDtypeStruct((N, EMB), jnp.float32),
    )(h, SR, MS0, MS1, CNT0, CNT1, K3, Wstack, root,
      bias.reshape(1, EMB), nnroot, nnb.reshape(1, EMB))


def kernel(x, edge_index, edge_attr, W_in, b_in, rgcn_weight, rgcn_root,
           rgcn_bias, nn_root, nn_bias, mlp_w1, mlp_b1, mlp_w2, mlp_b2):
    src = edge_index[0]
    dst = edge_index[1]
    etype = edge_attr[:, 1].astype(jnp.int32)
    seg_rel = dst * R + etype

    col = jnp.arange(16 * EMB)
    K1 = (jnp.arange(EMB)[:, None] == (col // EMB)[None, :]).astype(jnp.float32)
    K2 = (jnp.arange(EMB)[:, None] == (col % EMB)[None, :]).astype(jnp.float32)
    colr = jnp.arange(R * EMB)
    K3 = (jnp.arange(EMB)[:, None] == (colr // EMB)[None, :]).astype(jnp.float32)
    W2p = mlp_w2.reshape(EMB, EMB, EMB).transpose(1, 0, 2).reshape(EMB * EMB, EMB)
    B2 = mlp_b2.reshape(EMB, EMB)
    I8 = jnp.eye(8, dtype=jnp.float32)
    K1B = jnp.kron(I8, K1)
    K2B = jnp.kron(I8, K2)
    W2PB = jnp.kron(I8, W2p)
    B2B = jnp.kron(I8, B2)

    h = _node_proj(x, W_in, b_in)
    hidden, onehot = _edge_prep(edge_attr, mlp_w1, mlp_b1)

    SP_N = 10240
    src2 = src.reshape(NCH, CH)
    idx_ranged = jnp.stack([
        jnp.where(seg_rel < HNR, seg_rel, HNR),
        jnp.where(seg_rel >= HNR, seg_rel - HNR, HNR),
    ]).reshape(NC, NCH, CH)

    dst2 = dst.reshape(NCH, CH)
    cntp = _sc_scatter_add(onehot, dst2, SP_N)
    CNT0, CNT1 = cntp[0, :N], cntp[1, :N]
    hidden_p = hidden.reshape(E8, 128)

    for l in range(L):
        h_src = _sc_gather(h, src2)
        srp = _sc_sr_scatter(h_src, idx_ranged)
        SR = srp.reshape(N, R * EMB)
        msg_p = _msg(h_src.reshape(E8, 128), hidden_p, K1B, K2B, W2PB, B2B)
        dep = (srp[0, 0] * 0.0).astype(jnp.int32)[0]
        msp = _sc_scatter_add(msg_p.reshape(E, EMB), dst2 + dep, SP_N)
        MS0, MS1 = msp[0, :N], msp[1, :N]
        Wstack = rgcn_weight[l].reshape(R * EMB, EMB)
        h = _node_update(h, SR, MS0, MS1, CNT0, CNT1, K3, Wstack,
                         rgcn_root[l], rgcn_bias[l], nn_root[l], nn_bias[l])
    return h
